# baseline (speedup 1.0000x reference)
import functools

import jax
import jax.numpy as jnp
from jax import lax
from jax.experimental import pallas as pl
from jax.experimental.pallas import tpu as pltpu
from jax.experimental.pallas import tpu_sc as plsc

_B = 4096
_F = 26
_E = 16
_BF = _B * _F
_NW = 32
_PW = _BF // _NW
_CL = 128
_NCH = _PW // _CL
_BB = 512
_H = 400
_D_IN = _F * _E
_INV_STD = 0.9999950000374997
_V = 1000000
_NBLK = 7813
_DR = _V * _E // _CL


def _sc_detile(emb_t):
    mesh = plsc.VectorSubcoreMesh(core_axis_name="c", subcore_axis_name="s")

    @functools.partial(
        pl.kernel,
        mesh=mesh,
        out_type=jax.ShapeDtypeStruct((_DR, _CL), jnp.float32),
        scratch_types=[
            pltpu.VMEM((4, _E, _CL + 1), jnp.float32),
            pltpu.VMEM((4, _E, _CL), jnp.float32),
            pltpu.SemaphoreType.DMA,
            pltpu.SemaphoreType.DMA,
        ],
        compiler_params=pltpu.CompilerParams(
            use_tc_tiling_on_sc=True, needs_layout_passes=False),
    )
    def detile_kernel(embt_hbm, d_hbm, inbuf, outbuf, sin, sout):
        wid = lax.axis_index("s") * 2 + lax.axis_index("c")
        nblk_w = 244 + jnp.where(wid < _NBLK - 244 * _NW, 1, 0)

        def col0(b):
            return pl.multiple_of(jnp.where(b < _NBLK - 1, b * _CL, _V - _CL),
                                  _CL)

        def row0(b):
            return jnp.where(b < _NBLK - 1, _E * b, _DR - _E)

        def blk_of(t):
            return wid + _NW * t

        def fire_in(t, slot):
            pltpu.async_copy(
                embt_hbm.at[:, pl.ds(col0(blk_of(t)), _CL)],
                inbuf.at[slot, :, pl.ds(0, _CL)], sin)

        def in_copy(t, slot):
            return pltpu.make_async_copy(
                embt_hbm.at[:, pl.ds(col0(blk_of(t)), _CL)],
                inbuf.at[slot, :, pl.ds(0, _CL)], sin)

        def out_copy(t, slot):
            return pltpu.make_async_copy(
                outbuf.at[slot],
                d_hbm.at[pl.ds(row0(blk_of(t)), _E), :], sout)

        for s in range(4):
            fire_in(s, s)

        def group(g, carry):
            for s in range(4):
                t = g * 4 + s
                b = blk_of(t)

                @pl.when(b < _NBLK)
                def _():
                    in_copy(t, s).wait()

                    @pl.when(t >= 4)
                    def _():
                        out_copy(t - 4, s).wait()

                    blk = inbuf.at[s]
                    for m in range(_E):
                        for k in range(_CL // _E):
                            vals = plsc.load_gather(
                                blk, [lax.iota(jnp.int32, 16),
                                      jnp.full((16,), 8 * m + k, jnp.int32)])
                            outbuf[s, m, pl.ds(_E * k, _E)] = vals
                    out_copy(t, s).start()

                    @pl.when(blk_of(t + 4) < _NBLK)
                    def _():
                        fire_in(t + 4, s)
            return carry

        lax.fori_loop(0, 62, group, 0)

        for d in range(4):
            @pl.when(nblk_w - 4 + d >= 0)
            def _():
                out_copy(nblk_w - 4 + d, (nblk_w - 4 + d) % 4).wait()

    return detile_kernel(emb_t)


def _sc_gather_e(d_tab, idx_h3, idx_l3):
    mesh = plsc.VectorSubcoreMesh(core_axis_name="c", subcore_axis_name="s")

    @functools.partial(
        pl.kernel,
        mesh=mesh,
        out_type=jax.ShapeDtypeStruct((_BF * _E // _CL, _CL), jnp.float32),
        scratch_types=[
            pltpu.VMEM((_NCH, _CL), jnp.int32),
            pltpu.VMEM((_NCH, _CL), jnp.int32),
            pltpu.VMEM((2, _CL, _CL), jnp.float32),
            pltpu.VMEM((_PW * _E // _CL, _CL), jnp.float32),
            pltpu.VMEM((_E, _E), jnp.float32),
            pltpu.SemaphoreType.DMA,
        ],
        compiler_params=pltpu.CompilerParams(
            use_tc_tiling_on_sc=True, needs_layout_passes=False),
    )
    def gather_kernel(d_hbm, idxh_hbm, idxl_hbm, out_e,
                      idxh_v, idxl_v, staged, erows, tbuf, sem):
        wid = lax.axis_index("s") * 2 + lax.axis_index("c")
        pltpu.sync_copy(idxh_hbm.at[wid], idxh_v)
        pltpu.sync_copy(idxl_hbm.at[wid], idxl_v)

        def fire(c, slot):
            pltpu.async_copy(d_hbm.at[idxh_v.at[c]], staged.at[slot], sem)

        def drain(c, slot):
            pltpu.make_async_copy(
                d_hbm.at[idxh_v.at[c]], staged.at[slot], sem).wait()

        fire(0, 0)
        iota16 = lax.iota(jnp.int32, 16)

        def chunk(c, carry):
            @pl.when(c + 1 < _NCH)
            def _():
                fire(c + 1, (c + 1) % 2)
            drain(c, c % 2)

            for k in range(_CL // _E):
                off16 = idxl_v[c, pl.ds(_E * k, _E)] * _E
                rows16 = iota16 + _E * k
                for co in range(_E):
                    vals = plsc.load_gather(
                        staged.at[c % 2], [rows16, off16 + co])
                    plsc.store_scatter(
                        tbuf, [iota16, jnp.full((16,), co, jnp.int32)], vals)
                for r in range(_E):
                    erows[16 * c + 2 * k + r // 8,
                          pl.ds((r % 8) * _E, _E)] = tbuf[r, :]
            return carry

        lax.fori_loop(0, _NCH, chunk, 0)
        pltpu.sync_copy(erows, out_e.at[pl.ds(wid * (_PW * _E // _CL),
                                              _PW * _E // _CL)])

    return gather_kernel(d_tab, idx_h3, idx_l3)


def _sc_gather_f(fw16, idx_hi, idx_lo):
    mesh = plsc.VectorSubcoreMesh(core_axis_name="c", subcore_axis_name="s")

    @functools.partial(
        pl.kernel,
        mesh=mesh,
        out_type=jax.ShapeDtypeStruct((_NW, _PW), jnp.float32),
        scratch_types=[
            pltpu.VMEM((_NCH, _CL), jnp.int32),
            pltpu.VMEM((_NCH, _CL), jnp.int32),
            pltpu.VMEM((_PW, _E), jnp.float32),
            pltpu.VMEM((_PW,), jnp.float32),
            pltpu.SemaphoreType.DMA,
        ],
        compiler_params=pltpu.CompilerParams(
            use_tc_tiling_on_sc=False, needs_layout_passes=False),
    )
    def gatherf_kernel(fw16_hbm, idxhi_hbm, idxlo_hbm, out_f,
                       idxhi_v, idxlo_v, frows16, fwvals, sem):
        wid = lax.axis_index("s") * 2 + lax.axis_index("c")
        pltpu.sync_copy(idxhi_hbm.at[wid], idxhi_v)
        pltpu.sync_copy(idxlo_hbm.at[wid], idxlo_v)

        def fire(c, carry):
            pltpu.async_copy(fw16_hbm.at[idxhi_v.at[c]],
                             frows16.at[pl.ds(c * _CL, _CL)], sem)
            return carry

        lax.fori_loop(0, _NCH, fire, 0)

        def drain(c, carry):
            pltpu.make_async_copy(
                fw16_hbm.at[idxhi_v.at[c]],
                frows16.at[pl.ds(c * _CL, _CL)], sem).wait()
            return carry

        lax.fori_loop(0, _NCH, drain, 0)

        lane_iota = lax.iota(jnp.int32, 16)

        def select(k, carry):
            c = k // (_CL // 16)
            j = k - c * (_CL // 16)
            lane = idxlo_v[c, pl.ds(j * 16, 16)]
            rows = lane_iota + k * 16
            fwvals[pl.ds(k * 16, 16)] = plsc.load_gather(frows16, [rows, lane])
            return carry

        lax.fori_loop(0, _PW // 16, select, 0)
        pltpu.sync_copy(fwvals, out_f.at[wid])

    return gatherf_kernel(fw16, idx_hi, idx_lo)


def _tc_body(e_ref, fw_ref, fv_ref,
             w1_ref, b1_ref, g1_ref, be1_ref,
             w2_ref, b2_ref, g2_ref, be2_ref,
             w3_ref, b3_ref, g3_ref, be3_ref,
             wfc_ref, bfc_ref, out_ref):
    fv = fv_ref[...]
    fw = fw_ref[...]
    e_raw = e_ref[...]

    rep_f = lax.broadcasted_iota(jnp.int32, (_F, _D_IN), 0)
    rep_j = lax.broadcasted_iota(jnp.int32, (_F, _D_IN), 1) // _E
    rep = (rep_f == rep_j).astype(jnp.float32)
    fvx = jnp.dot(fv, rep, preferred_element_type=jnp.float32,
                  precision=lax.Precision.HIGHEST)
    e = e_raw * fvx

    sum_j = lax.broadcasted_iota(jnp.int32, (_D_IN, _E), 0) % _E
    sum_k = lax.broadcasted_iota(jnp.int32, (_D_IN, _E), 1)
    smat = (sum_j == sum_k).astype(jnp.float32)
    summed = jnp.dot(e, smat, preferred_element_type=jnp.float32,
                     precision=lax.Precision.HIGHEST)
    sumsq = jnp.dot(e * e, smat, preferred_element_type=jnp.float32,
                    precision=lax.Precision.HIGHEST)
    y_secd = 0.5 * (summed * summed - sumsq)

    y_first = fw * fv

    def dot16(a, b):
        return jnp.dot(a.astype(jnp.bfloat16), b.astype(jnp.bfloat16),
                       preferred_element_type=jnp.float32)

    h = dot16(e, w1_ref[...]) + b1_ref[...]
    h = jnp.maximum(h * (_INV_STD * g1_ref[...]) + be1_ref[...], 0.0)
    h = dot16(h, w2_ref[...]) + b2_ref[...]
    h = jnp.maximum(h * (_INV_STD * g2_ref[...]) + be2_ref[...], 0.0)
    h = dot16(h, w3_ref[...]) + b3_ref[...]
    h = jnp.maximum(h * (_INV_STD * g3_ref[...]) + be3_ref[...], 0.0)

    wfc = wfc_ref[...]
    out = (dot16(y_first, wfc[0:_F, :])
           + dot16(y_secd, wfc[_F:_F + _E, :])
           + dot16(h, wfc[_F + _E:, :])
           + bfc_ref[...])
    out_ref[...] = out


def _tc_dense(e_raw, fw, fv, W1, b1, g1, be1, W2, b2, g2, be2,
              W3, b3, g3, be3, Wfc, bfc):
    grid = (_B // _BB,)

    def row_block(i):
        return (i, 0)

    def whole(i):
        return (0, 0)

    bspec = lambda shape, imap: pl.BlockSpec(shape, imap)
    in_specs = [
        bspec((_BB, _D_IN), row_block),
        bspec((_BB, _F), row_block),
        bspec((_BB, _F), row_block),
        bspec((_D_IN, _H), whole), bspec((1, _H), whole),
        bspec((1, _H), whole), bspec((1, _H), whole),
        bspec((_H, _H), whole), bspec((1, _H), whole),
        bspec((1, _H), whole), bspec((1, _H), whole),
        bspec((_H, _H), whole), bspec((1, _H), whole),
        bspec((1, _H), whole), bspec((1, _H), whole),
        bspec((_F + _E + _H, 1), whole), bspec((1, 1), whole),
    ]
    return pl.pallas_call(
        _tc_body,
        grid=grid,
        in_specs=in_specs,
        out_specs=pl.BlockSpec((_BB, 1), row_block),
        out_shape=jax.ShapeDtypeStruct((_B, 1), jnp.float32),
    )(e_raw, fw, fv, W1, b1, g1, be1, W2, b2, g2, be2,
      W3, b3, g3, be3, Wfc, bfc)


def kernel(feat_index, feat_value, first_w, emb,
           W1, b1, g1, be1, W2, b2, g2, be2, W3, b3, g3, be3,
           Wfc, bfc):
    idx = feat_index.astype(jnp.int32).reshape(_NW, _NCH, _CL)
    d_tab = _sc_detile(emb.T)
    e_rows = _sc_gather_e(d_tab, idx >> 3, idx & 7)
    f_rows = _sc_gather_f(first_w.reshape(-1, _E), idx >> 4, idx & 15)
    e_raw = e_rows.reshape(_B, _D_IN)
    fw = f_rows.reshape(_B, _F)
    out = _tc_dense(
        e_raw, fw, feat_value,
        W1, b1.reshape(1, _H), g1.reshape(1, _H), be1.reshape(1, _H),
        W2, b2.reshape(1, _H), g2.reshape(1, _H), be2.reshape(1, _H),
        W3, b3.reshape(1, _H), g3.reshape(1, _H), be3.reshape(1, _H),
        Wfc, bfc.reshape(1, 1))
    return out


# speedup vs baseline: 1.1537x; 1.1537x over previous
import functools

import jax
import jax.numpy as jnp
from jax import lax
from jax.experimental import pallas as pl
from jax.experimental.pallas import tpu as pltpu
from jax.experimental.pallas import tpu_sc as plsc

_B = 4096
_F = 26
_E = 16
_BF = _B * _F
_NW = 32
_PW = _BF // _NW
_CL = 128
_NCH = _PW // _CL
_BB = 512
_H = 400
_D_IN = _F * _E
_INV_STD = 0.9999950000374997


def _sc_gather(emb, fw16, idx_r, idx_hi, idx_lo):
    mesh = plsc.VectorSubcoreMesh(core_axis_name="c", subcore_axis_name="s")

    @functools.partial(
        pl.kernel,
        mesh=mesh,
        out_type=[
            jax.ShapeDtypeStruct((_NW, _PW, _E), jnp.float32),
            jax.ShapeDtypeStruct((_NW, _PW), jnp.float32),
        ],
        scratch_types=[
            pltpu.VMEM((_NCH, _CL), jnp.int32),
            pltpu.VMEM((_NCH, _CL), jnp.int32),
            pltpu.VMEM((_NCH, _CL), jnp.int32),
            pltpu.VMEM((_PW, _E), jnp.float32),
            pltpu.VMEM((_PW, _E), jnp.float32),
            pltpu.VMEM((_PW,), jnp.float32),
            pltpu.SemaphoreType.DMA,
        ],
        compiler_params=pltpu.CompilerParams(
            use_tc_tiling_on_sc=False, needs_layout_passes=False),
    )
    def gather_kernel(emb_hbm, fw16_hbm, idx_hbm, idxhi_hbm, idxlo_hbm,
                      out_e, out_f,
                      idx_v, idxhi_v, idxlo_v, erows, frows16, fwvals, sem):
        wid = lax.axis_index("s") * 2 + lax.axis_index("c")
        pltpu.sync_copy(idx_hbm.at[wid], idx_v)
        pltpu.sync_copy(idxhi_hbm.at[wid], idxhi_v)
        pltpu.sync_copy(idxlo_hbm.at[wid], idxlo_v)

        def fire(c, carry):
            pltpu.async_copy(emb_hbm.at[idx_v.at[c]],
                             erows.at[pl.ds(c * _CL, _CL)], sem)
            pltpu.async_copy(fw16_hbm.at[idxhi_v.at[c]],
                             frows16.at[pl.ds(c * _CL, _CL)], sem)
            return carry

        lax.fori_loop(0, _NCH, fire, 0)

        def drain(c, carry):
            pltpu.make_async_copy(
                emb_hbm.at[idx_v.at[c]],
                erows.at[pl.ds(c * _CL, _CL)], sem).wait()
            pltpu.make_async_copy(
                fw16_hbm.at[idxhi_v.at[c]],
                frows16.at[pl.ds(c * _CL, _CL)], sem).wait()
            return carry

        lax.fori_loop(0, _NCH, drain, 0)
        pltpu.sync_copy(erows, out_e.at[wid])

        lane_iota = lax.iota(jnp.int32, 16)

        def select(k, carry):
            c = k // (_CL // 16)
            j = k - c * (_CL // 16)
            lane = idxlo_v[c, pl.ds(j * 16, 16)]
            rows = lane_iota + k * 16
            fwvals[pl.ds(k * 16, 16)] = plsc.load_gather(frows16, [rows, lane])
            return carry

        lax.fori_loop(0, _PW // 16, select, 0)
        pltpu.sync_copy(fwvals, out_f.at[wid])

    return gather_kernel(emb, fw16, idx_r, idx_hi, idx_lo)


def _tc_body(e_ref, fw_ref, fv_ref,
             w1_ref, b1_ref, g1_ref, be1_ref,
             w2_ref, b2_ref, g2_ref, be2_ref,
             w3_ref, b3_ref, g3_ref, be3_ref,
             wfc_ref, bfc_ref, out_ref):
    fv = fv_ref[...]
    fw = fw_ref[...]
    e_raw = e_ref[...]

    rep_f = lax.broadcasted_iota(jnp.int32, (_F, _D_IN), 0)
    rep_j = lax.broadcasted_iota(jnp.int32, (_F, _D_IN), 1) // _E
    rep = (rep_f == rep_j).astype(jnp.float32)
    fvx = jnp.dot(fv, rep, preferred_element_type=jnp.float32, precision=lax.Precision.HIGHEST)
    e = e_raw * fvx

    sum_j = lax.broadcasted_iota(jnp.int32, (_D_IN, _E), 0) % _E
    sum_k = lax.broadcasted_iota(jnp.int32, (_D_IN, _E), 1)
    smat = (sum_j == sum_k).astype(jnp.float32)
    summed = jnp.dot(e, smat, preferred_element_type=jnp.float32, precision=lax.Precision.HIGHEST)
    sumsq = jnp.dot(e * e, smat, preferred_element_type=jnp.float32, precision=lax.Precision.HIGHEST)
    y_secd = 0.5 * (summed * summed - sumsq)

    y_first = fw * fv

    def dot16(a, b):
        return jnp.dot(a.astype(jnp.bfloat16), b.astype(jnp.bfloat16),
                       preferred_element_type=jnp.float32)

    h = dot16(e, w1_ref[...]) + b1_ref[...]
    h = jnp.maximum(h * (_INV_STD * g1_ref[...]) + be1_ref[...], 0.0)
    h = dot16(h, w2_ref[...]) + b2_ref[...]
    h = jnp.maximum(h * (_INV_STD * g2_ref[...]) + be2_ref[...], 0.0)
    h = dot16(h, w3_ref[...]) + b3_ref[...]
    h = jnp.maximum(h * (_INV_STD * g3_ref[...]) + be3_ref[...], 0.0)

    wfc = wfc_ref[...]
    out = (dot16(y_first, wfc[0:_F, :])
           + dot16(y_secd, wfc[_F:_F + _E, :])
           + dot16(h, wfc[_F + _E:, :])
           + bfc_ref[...])
    out_ref[...] = out


def _tc_dense(e_raw, fw, fv, W1, b1, g1, be1, W2, b2, g2, be2,
              W3, b3, g3, be3, Wfc, bfc):
    grid = (_B // _BB,)

    def row_block(i):
        return (i, 0)

    def whole(i):
        return (0, 0)

    bspec = lambda shape, imap: pl.BlockSpec(shape, imap)
    in_specs = [
        bspec((_BB, _D_IN), row_block),
        bspec((_BB, _F), row_block),
        bspec((_BB, _F), row_block),
        bspec((_D_IN, _H), whole), bspec((1, _H), whole),
        bspec((1, _H), whole), bspec((1, _H), whole),
        bspec((_H, _H), whole), bspec((1, _H), whole),
        bspec((1, _H), whole), bspec((1, _H), whole),
        bspec((_H, _H), whole), bspec((1, _H), whole),
        bspec((1, _H), whole), bspec((1, _H), whole),
        bspec((_F + _E + _H, 1), whole), bspec((1, 1), whole),
    ]
    return pl.pallas_call(
        _tc_body,
        grid=grid,
        in_specs=in_specs,
        out_specs=pl.BlockSpec((_BB, 1), row_block),
        out_shape=jax.ShapeDtypeStruct((_B, 1), jnp.float32),
    )(e_raw, fw, fv, W1, b1, g1, be1, W2, b2, g2, be2,
      W3, b3, g3, be3, Wfc, bfc)


def kernel(feat_index, feat_value, first_w, emb,
           W1, b1, g1, be1, W2, b2, g2, be2, W3, b3, g3, be3,
           Wfc, bfc):
    idx_r = feat_index.astype(jnp.int32).reshape(_NW, _NCH, _CL)
    e_rows, f_rows = _sc_gather(emb, first_w.reshape(-1, _E), idx_r,
                                idx_r >> 4, idx_r & 15)
    e_raw = e_rows.reshape(_B, _D_IN)
    fw = f_rows.reshape(_B, _F)
    out = _tc_dense(
        e_raw, fw, feat_value,
        W1, b1.reshape(1, _H), g1.reshape(1, _H), be1.reshape(1, _H),
        W2, b2.reshape(1, _H), g2.reshape(1, _H), be2.reshape(1, _H),
        W3, b3.reshape(1, _H), g3.reshape(1, _H), be3.reshape(1, _H),
        Wfc, bfc.reshape(1, 1))
    return out
